# Initial kernel scaffold; baseline (speedup 1.0000x reference)
import jax
import jax.numpy as jnp
from jax.experimental import pallas as pl


def kernel(X, edge_index, edge_weight, W1, W2):
    raise NotImplementedError("write your pallas kernel here")


# speedup vs baseline: 1.9909x; 1.9909x over previous
import functools

import jax
import jax.numpy as jnp
from jax.experimental import pallas as pl


N = 10000
D_IN = 256
D_MID = 256
D_EMB = 64


def _mm_body(x_ref, w_ref, o_ref, *, relu_in):
    x = x_ref[...]
    if relu_in:
        x = jnp.maximum(x, 0.0)
    o_ref[...] = jax.lax.dot_general(
        x, w_ref[...], (((1,), (0,)), ((), ())),
        preferred_element_type=jnp.float32)


def _matmul(x, w, relu_in=False, br=1000):
    m, k = x.shape
    k2, n = w.shape
    grid = m // br
    return pl.pallas_call(
        functools.partial(_mm_body, relu_in=relu_in),
        grid=(grid,),
        in_specs=[
            pl.BlockSpec((br, k), lambda i: (i, 0)),
            pl.BlockSpec((k, n), lambda i: (0, 0)),
        ],
        out_specs=pl.BlockSpec((br, n), lambda i: (i, 0)),
        out_shape=jax.ShapeDtypeStruct((m, n), jnp.float32),
    )(x, w)


def _dist_body(eb_ref, ea_ref, o_ref):
    eb = eb_ref[...]
    ea = ea_ref[...]
    sqa = jnp.sum(ea * ea, axis=1, keepdims=True)
    ea_aug = jnp.concatenate([ea, -sqa], axis=1)
    ones = jnp.ones((eb.shape[0], 1), jnp.float32)
    eb_aug = jnp.concatenate([2.0 * eb, ones], axis=1)
    t0 = jax.lax.dot_general(
        eb_aug, ea_aug, (((1,), (1,)), ((), ())),
        preferred_element_type=jnp.float32)
    sqb = jnp.sum(eb * eb, axis=1, keepdims=True)
    t = jnp.minimum(t0 - sqb, 0.0)
    m = jnp.max(t, axis=1, keepdims=True)
    e = jnp.exp(t - m)
    s = jnp.sum(e, axis=1, keepdims=True)
    o_ref[...] = e / s + 1e-10


def _dist_softmax(emb, br=200):
    n, d = emb.shape
    grid = n // br
    return pl.pallas_call(
        _dist_body,
        grid=(grid,),
        in_specs=[
            pl.BlockSpec((br, d), lambda i: (i, 0)),
            pl.BlockSpec((n, d), lambda i: (0, 0)),
        ],
        out_specs=pl.BlockSpec((br, n), lambda i: (i, 0)),
        out_shape=jax.ShapeDtypeStruct((n, n), jnp.float32),
    )(emb, emb)


def _spmm(edge_index, edge_weight, M):
    src = edge_index[0]
    dst = edge_index[1]
    msgs = jnp.take(M, src, axis=0) * edge_weight[:, None]
    return jnp.zeros((N, M.shape[1]), dtype=M.dtype).at[dst].add(msgs)


def kernel(X, edge_index, edge_weight, W1, W2):
    xw1 = _matmul(X, W1)
    h = _spmm(edge_index, edge_weight, xw1)
    hw2 = _matmul(h, W2, relu_in=True)
    emb = _spmm(edge_index, edge_weight, hw2)
    return _dist_softmax(emb)


# speedup vs baseline: 2.2731x; 1.1418x over previous
import functools

import jax
import jax.numpy as jnp
from jax import lax
from jax.experimental import pallas as pl
from jax.experimental.pallas import tpu as pltpu
from jax.experimental.pallas import tpu_sc as plsc


N = 10000
E = 160000
D_IN = 256
D_MID = 256
D_EMB = 64

_NC, _NS = 2, 16
_NW = _NC * _NS


def _mm_body(x_ref, w_ref, o_ref, *, relu_in):
    x = x_ref[...]
    if relu_in:
        x = jnp.maximum(x, 0.0)
    o_ref[...] = jax.lax.dot_general(
        x, w_ref[...], (((1,), (0,)), ((), ())),
        preferred_element_type=jnp.float32)


def _matmul(x, w, relu_in=False, br=1000):
    m, k = x.shape
    k2, n = w.shape
    grid = m // br
    return pl.pallas_call(
        functools.partial(_mm_body, relu_in=relu_in),
        grid=(grid,),
        in_specs=[
            pl.BlockSpec((br, k), lambda i: (i, 0)),
            pl.BlockSpec((k, n), lambda i: (0, 0)),
        ],
        out_specs=pl.BlockSpec((br, n), lambda i: (i, 0)),
        out_shape=jax.ShapeDtypeStruct((m, n), jnp.float32),
    )(x, w)


def _make_spmm_sc(n, d, e, splits, chunk, sub):
    g_groups = d // 8
    assert g_groups * splits == _NW
    ept = e // splits
    nchunks = ept // chunk
    nsub = chunk // sub
    assert nchunks * chunk == ept and nsub * sub == chunk and sub % 16 == 0
    mesh = plsc.VectorSubcoreMesh(core_axis_name="c", subcore_axis_name="s",
                                  num_cores=_NC, num_subcores=_NS)

    @functools.partial(
        pl.kernel,
        out_type=jax.ShapeDtypeStruct((splits, n, d), jnp.float32),
        mesh=mesh,
        scratch_types=[
            pltpu.VMEM((chunk,), jnp.int32),
            pltpu.VMEM((nsub, sub), jnp.int32),
            pltpu.VMEM((chunk, 8), jnp.float32),
            pltpu.VMEM((chunk * 8,), jnp.int32),
            pltpu.VMEM((chunk * 8,), jnp.float32),
            pltpu.VMEM((n, 8), jnp.float32),
            pltpu.SemaphoreType.DMA,
            pltpu.SemaphoreType.DMA,
            pltpu.SemaphoreType.DMA,
            pltpu.SemaphoreType.DMA,
        ],
        compiler_params=pltpu.CompilerParams(use_tc_tiling_on_sc=False,
                                             needs_layout_passes=False),
    )
    def spmm(m_hbm, srcg_hbm, dstr_hbm, wx_hbm, out_hbm,
             srcg_v, gidx_v, rows_v, dstr_v, wx_v, acc_v,
             sem0, sem1, sem2, sem3):
        wid = lax.axis_index("s") * _NC + lax.axis_index("c")
        g = wid % g_groups
        sp = wid // g_groups
        ebase = sp * ept
        zero16 = jnp.zeros((16,), jnp.float32)
        gsplat = jnp.full((16,), g, jnp.int32)
        col16 = lax.iota(jnp.int32, 16) & 7
        pair16 = lax.iota(jnp.int32, 16) >> 3

        def zbody(i, _):
            plsc.store_scatter(acc_v, [pair16 + 2 * i, col16], zero16)
            return ()
        lax.fori_loop(0, n // 2, zbody, (), unroll=8)

        def chunk_body(ci, _):
            eb = ebase + ci * chunk
            cp0 = pltpu.make_async_copy(
                srcg_hbm.at[pl.ds(eb, chunk)], srcg_v, sem0)
            cp0.start()
            cp1 = pltpu.make_async_copy(
                dstr_hbm.at[pl.ds(eb * 8, chunk * 8)], dstr_v, sem1)
            cp1.start()
            cp2 = pltpu.make_async_copy(
                wx_hbm.at[pl.ds(eb * 8, chunk * 8)], wx_v, sem2)
            cp2.start()
            cp0.wait()

            for j in range(nsub):
                def gibody(q, _, j=j):
                    sl16 = pl.ds(j * sub + q * 16, 16)
                    gidx_v[j, pl.ds(q * 16, 16)] = srcg_v[sl16] + gsplat
                    return ()
                lax.fori_loop(0, sub // 16, gibody, (), unroll=4)
            gcps = []
            for j in range(nsub):
                gcp = pltpu.make_async_copy(
                    m_hbm.at[gidx_v.at[j]],
                    rows_v.at[pl.ds(j * sub, sub)], sem3)
                gcp.start()
                gcps.append(gcp)
            for gcp in gcps:
                gcp.wait()
            cp1.wait()
            cp2.wait()

            def pbody(p, _):
                sl = pl.ds(p * 16, 16)
                x = plsc.load_gather(rows_v, [pair16 + 2 * p, col16])
                x = x * wx_v[sl]
                plsc.addupdate_scatter(acc_v, [dstr_v[sl], col16], x)
                return ()
            lax.fori_loop(0, chunk // 2, pbody, (), unroll=8)
            return ()

        lax.fori_loop(0, nchunks, chunk_body, ())

        pltpu.sync_copy(acc_v, out_hbm.at[sp].at[:, pl.ds(g * 8, 8)])

    return spmm


_spmm1 = _make_spmm_sc(N, D_MID, E, splits=1, chunk=1280, sub=64)
_spmm2 = _make_spmm_sc(N, D_EMB, E, splits=4, chunk=1600, sub=64)


def _merge_body(x_ref, o_ref):
    o_ref[...] = jnp.sum(x_ref[...], axis=0)


def _merge(parts, br=1000):
    s, n, d = parts.shape
    return pl.pallas_call(
        _merge_body,
        grid=(n // br,),
        in_specs=[pl.BlockSpec((s, br, d), lambda i: (0, i, 0))],
        out_specs=pl.BlockSpec((br, d), lambda i: (i, 0)),
        out_shape=jax.ShapeDtypeStruct((n, d), jnp.float32),
    )(parts)


def _dist_body(eb_ref, ea_ref, o_ref):
    eb = eb_ref[...]
    ea = ea_ref[...]
    sqa = jnp.sum(ea * ea, axis=1, keepdims=True)
    ea_aug = jnp.concatenate([ea, -sqa], axis=1)
    ones = jnp.ones((eb.shape[0], 1), jnp.float32)
    eb_aug = jnp.concatenate([2.0 * eb, ones], axis=1)
    t0 = jax.lax.dot_general(
        eb_aug, ea_aug, (((1,), (1,)), ((), ())),
        preferred_element_type=jnp.float32)
    sqb = jnp.sum(eb * eb, axis=1, keepdims=True)
    t = jnp.minimum(t0 - sqb, 0.0)
    m = jnp.max(t, axis=1, keepdims=True)
    ex = jnp.exp(t - m)
    s = jnp.sum(ex, axis=1, keepdims=True)
    o_ref[...] = ex / s + 1e-10


def _dist_softmax(emb, br=200):
    n, d = emb.shape
    grid = n // br
    return pl.pallas_call(
        _dist_body,
        grid=(grid,),
        in_specs=[
            pl.BlockSpec((br, d), lambda i: (i, 0)),
            pl.BlockSpec((n, d), lambda i: (0, 0)),
        ],
        out_specs=pl.BlockSpec((br, n), lambda i: (i, 0)),
        out_shape=jax.ShapeDtypeStruct((n, n), jnp.float32),
    )(emb, emb)


def kernel(X, edge_index, edge_weight, W1, W2):
    src = edge_index[0]
    dst = edge_index[1]
    dstr = jnp.repeat(dst, 8)
    wx = jnp.repeat(edge_weight, 8)

    xw1 = _matmul(X, W1)
    h = _spmm1(xw1.reshape(N * (D_MID // 8), 8), src * (D_MID // 8),
               dstr, wx)[0]
    hw2 = _matmul(h, W2, relu_in=True)
    emb_parts = _spmm2(hw2.reshape(N * (D_EMB // 8), 8), src * (D_EMB // 8),
                       dstr, wx)
    emb = _merge(emb_parts)
    return _dist_softmax(emb)


# speedup vs baseline: 3.1308x; 1.3773x over previous
import functools

import jax
import jax.numpy as jnp
from jax import lax
from jax.experimental import pallas as pl
from jax.experimental.pallas import tpu as pltpu
from jax.experimental.pallas import tpu_sc as plsc


N = 10000
E = 160000
D_IN = 256
D_MID = 256
D_EMB = 64

_NC, _NS = 2, 16
_NW = _NC * _NS


def _mm_body(x_ref, w_ref, o_ref, *, relu_in):
    x = x_ref[...]
    if relu_in:
        x = jnp.maximum(x, 0.0)
    o_ref[...] = jax.lax.dot_general(
        x, w_ref[...], (((1,), (0,)), ((), ())),
        preferred_element_type=jnp.float32)


def _matmul(x, w, relu_in=False, br=1000):
    m, k = x.shape
    k2, n = w.shape
    grid = m // br
    return pl.pallas_call(
        functools.partial(_mm_body, relu_in=relu_in),
        grid=(grid,),
        in_specs=[
            pl.BlockSpec((br, k), lambda i: (i, 0)),
            pl.BlockSpec((k, n), lambda i: (0, 0)),
        ],
        out_specs=pl.BlockSpec((br, n), lambda i: (i, 0)),
        out_shape=jax.ShapeDtypeStruct((m, n), jnp.float32),
    )(x, w)


def _make_spmm_sc(n, d, e, splits, chunk, sub):
    g_groups = d // 8
    assert g_groups * splits == _NW
    ept = e // splits
    nchunks = ept // chunk
    nsub = chunk // sub
    assert nchunks * chunk == ept and nsub * sub == chunk and sub % 16 == 0
    mesh = plsc.VectorSubcoreMesh(core_axis_name="c", subcore_axis_name="s",
                                  num_cores=_NC, num_subcores=_NS)

    @functools.partial(
        pl.kernel,
        out_type=jax.ShapeDtypeStruct((splits, n, d), jnp.float32),
        mesh=mesh,
        scratch_types=[
            pltpu.VMEM((chunk,), jnp.int32),
            pltpu.VMEM((nsub, sub), jnp.int32),
            pltpu.VMEM((chunk, 8), jnp.float32),
            pltpu.VMEM((chunk * 8,), jnp.int32),
            pltpu.VMEM((chunk * 8,), jnp.float32),
            pltpu.VMEM((n, 8), jnp.float32),
            pltpu.SemaphoreType.DMA,
            pltpu.SemaphoreType.DMA,
            pltpu.SemaphoreType.DMA,
            pltpu.SemaphoreType.DMA,
        ],
        compiler_params=pltpu.CompilerParams(use_tc_tiling_on_sc=False,
                                             needs_layout_passes=False),
    )
    def spmm(m_hbm, srcg_hbm, dstr_hbm, wx_hbm, out_hbm,
             srcg_v, gidx_v, rows_v, dstr_v, wx_v, acc_v,
             sem0, sem1, sem2, sem3):
        wid = lax.axis_index("s") * _NC + lax.axis_index("c")
        g = wid % g_groups
        sp = wid // g_groups
        ebase = sp * ept
        zero16 = jnp.zeros((16,), jnp.float32)
        gsplat = jnp.full((16,), g, jnp.int32)
        col16 = lax.iota(jnp.int32, 16) & 7
        pair16 = lax.iota(jnp.int32, 16) >> 3

        @plsc.parallel_loop(0, n // 2, unroll=8)
        def _(i):
            plsc.store_scatter(acc_v, [pair16 + 2 * i, col16], zero16)

        def chunk_body(ci, _):
            eb = ebase + ci * chunk
            cp0 = pltpu.make_async_copy(
                srcg_hbm.at[pl.ds(eb, chunk)], srcg_v, sem0)
            cp0.start()
            cp1 = pltpu.make_async_copy(
                dstr_hbm.at[pl.ds(eb * 8, chunk * 8)], dstr_v, sem1)
            cp1.start()
            cp2 = pltpu.make_async_copy(
                wx_hbm.at[pl.ds(eb * 8, chunk * 8)], wx_v, sem2)
            cp2.start()
            cp0.wait()

            for j in range(nsub):
                @plsc.parallel_loop(0, sub // 16, unroll=4)
                def _(q, j=j):
                    sl16 = pl.ds(j * sub + q * 16, 16)
                    gidx_v[j, pl.ds(q * 16, 16)] = srcg_v[sl16] + gsplat
            gcps = []
            for j in range(nsub):
                gcp = pltpu.make_async_copy(
                    m_hbm.at[gidx_v.at[j]],
                    rows_v.at[pl.ds(j * sub, sub)], sem3)
                gcp.start()
                gcps.append(gcp)
            for gcp in gcps:
                gcp.wait()
            cp1.wait()
            cp2.wait()

            @plsc.parallel_loop(0, chunk // 2, unroll=8)
            def _(p):
                sl = pl.ds(p * 16, 16)
                x = plsc.load_gather(rows_v, [pair16 + 2 * p, col16])
                x = x * wx_v[sl]
                plsc.addupdate_scatter(acc_v, [dstr_v[sl], col16], x)
            return ()

        lax.fori_loop(0, nchunks, chunk_body, ())

        pltpu.sync_copy(acc_v, out_hbm.at[sp].at[:, pl.ds(g * 8, 8)])

    return spmm


_spmm1 = _make_spmm_sc(N, D_MID, E, splits=1, chunk=1280, sub=64)
_spmm2 = _make_spmm_sc(N, D_EMB, E, splits=4, chunk=1600, sub=64)


def _merge_body(x_ref, o_ref):
    o_ref[...] = jnp.sum(x_ref[...], axis=0)


def _merge(parts, br=1000):
    s, n, d = parts.shape
    return pl.pallas_call(
        _merge_body,
        grid=(n // br,),
        in_specs=[pl.BlockSpec((s, br, d), lambda i: (0, i, 0))],
        out_specs=pl.BlockSpec((br, d), lambda i: (i, 0)),
        out_shape=jax.ShapeDtypeStruct((n, d), jnp.float32),
    )(parts)


def _dist_body(eb_ref, ea_ref, o_ref):
    eb = eb_ref[...]
    ea = ea_ref[...]
    sqa = jnp.sum(ea * ea, axis=1, keepdims=True)
    onesa = jnp.ones((ea.shape[0], 1), jnp.float32)
    ea_aug = jnp.concatenate([ea, -sqa, -onesa], axis=1)
    sqb = jnp.sum(eb * eb, axis=1, keepdims=True)
    onesb = jnp.ones((eb.shape[0], 1), jnp.float32)
    eb_aug = jnp.concatenate([2.0 * eb, onesb, sqb], axis=1)
    t0 = jax.lax.dot_general(
        eb_aug, ea_aug, (((1,), (1,)), ((), ())),
        preferred_element_type=jnp.float32)
    ex = jnp.exp(jnp.minimum(t0, 0.0))
    s = jnp.sum(ex, axis=1, keepdims=True)
    o_ref[...] = ex * (1.0 / s) + 1e-10


def _dist_softmax(emb, br=200):
    n, d = emb.shape
    grid = n // br
    return pl.pallas_call(
        _dist_body,
        grid=(grid,),
        in_specs=[
            pl.BlockSpec((br, d), lambda i: (i, 0)),
            pl.BlockSpec((n, d), lambda i: (0, 0)),
        ],
        out_specs=pl.BlockSpec((br, n), lambda i: (i, 0)),
        out_shape=jax.ShapeDtypeStruct((n, n), jnp.float32),
    )(emb, emb)


def kernel(X, edge_index, edge_weight, W1, W2):
    src = edge_index[0]
    dst = edge_index[1]
    dstr = jnp.repeat(dst, 8)
    wx = jnp.repeat(edge_weight, 8)

    xw1 = _matmul(X, W1)
    h = _spmm1(xw1.reshape(N * (D_MID // 8), 8), src * (D_MID // 8),
               dstr, wx)[0]
    hw2 = _matmul(h, W2, relu_in=True)
    emb_parts = _spmm2(hw2.reshape(N * (D_EMB // 8), 8), src * (D_EMB // 8),
                       dstr, wx)
    emb = _merge(emb_parts)
    return _dist_softmax(emb)
